# Initial kernel scaffold; baseline (speedup 1.0000x reference)
import jax
import jax.numpy as jnp
from jax.experimental import pallas as pl


def kernel(features, edge_index, W1, b1, W2, b2, Wm1, bm1, Wm2, bm2):
    raise NotImplementedError("write your pallas kernel here")


# speedup vs baseline: 32.1490x; 32.1490x over previous
import functools

import jax
import jax.numpy as jnp
from jax import lax
from jax.experimental import pallas as pl
from jax.experimental.pallas import tpu as pltpu
from jax.experimental.pallas import tpu_sc as plsc

N = 10000
NPAD = 10240
E = 320000
EPAD = 327680
DIN = 128
H = 64
HM = 128
OUT = 40
NC = 2
NS = 16
NW = NC * NS
CH = 512
CHUNKS = EPAD // (NW * CH)
IDX_ROWS = EPAD // 128
ROWS_PT = IDX_ROWS // NW
RPC = CH // 128
RPT = NPAD // NS
BLK = 512
GRID = NPAD // BLK

_mesh = plsc.VectorSubcoreMesh(core_axis_name="c", subcore_axis_name="s")


@functools.partial(
    pl.kernel,
    out_type=jax.ShapeDtypeStruct((NC, NPAD), jnp.float32),
    mesh=_mesh,
    scratch_types=[
        pltpu.VMEM((ROWS_PT, 128), jnp.int32),
        pltpu.VMEM((128,), jnp.float32),
        pltpu.VMEM_SHARED((NPAD,), jnp.float32),
    ],
)
def _deg_kernel(dst_hbm, zero_hbm, out_hbm, idx_v, ones_v, acc):
    c = lax.axis_index("c")
    s = lax.axis_index("s")
    wid = c * NS + s
    for i in range(8):
        ones_v[pl.ds(i * 16, 16)] = jnp.ones((16,), jnp.float32)
    pltpu.sync_copy(zero_hbm, acc.at[pl.ds(s * RPT, RPT)])
    pltpu.sync_copy(dst_hbm.at[pl.ds(wid * ROWS_PT, ROWS_PT)], idx_v)
    plsc.subcore_barrier()

    def body(j, carry):
        pltpu.sync_copy(ones_v, acc.at[idx_v.at[j]], add=True)
        return carry

    lax.fori_loop(0, ROWS_PT, body, 0)
    plsc.subcore_barrier()
    pltpu.sync_copy(acc.at[pl.ds(s * RPT, RPT)],
                    out_hbm.at[c, pl.ds(s * RPT, RPT)])


@functools.partial(
    pl.kernel,
    out_type=jax.ShapeDtypeStruct((NC, NPAD, H), jnp.float32),
    mesh=_mesh,
    compiler_params=pltpu.CompilerParams(use_tc_tiling_on_sc=False),
    scratch_types=[
        pltpu.VMEM((ROWS_PT, 128), jnp.int32),
        pltpu.VMEM((ROWS_PT, 128), jnp.int32),
        pltpu.VMEM((CH, H), jnp.float32),
        pltpu.VMEM_SHARED((NPAD, H), jnp.float32),
        pltpu.SemaphoreType.DMA,
    ],
)
def _prop_kernel(hp_hbm, src_hbm, dst_hbm, zrow_hbm, out_hbm,
                 src_v, dst_v, rows_v, acc, sem):
    c = lax.axis_index("c")
    s = lax.axis_index("s")
    wid = c * NS + s
    pltpu.sync_copy(zrow_hbm, acc.at[pl.ds(s * RPT, RPT)])
    pltpu.sync_copy(src_hbm.at[pl.ds(wid * ROWS_PT, ROWS_PT)], src_v)
    pltpu.sync_copy(dst_hbm.at[pl.ds(wid * ROWS_PT, ROWS_PT)], dst_v)
    plsc.subcore_barrier()

    def chunk(k, carry):
        base = k * RPC
        cps = [
            pltpu.async_copy(hp_hbm.at[src_v.at[base + j]],
                             rows_v.at[pl.ds(j * 128, 128)], sem)
            for j in range(RPC)
        ]
        for cp in cps:
            cp.wait()
        for j in range(RPC):
            pltpu.sync_copy(rows_v.at[pl.ds(j * 128, 128)],
                            acc.at[dst_v.at[base + j]], add=True)
        return carry

    lax.fori_loop(0, CHUNKS, chunk, 0)
    plsc.subcore_barrier()
    pltpu.sync_copy(acc.at[pl.ds(s * RPT, RPT)],
                    out_hbm.at[c, pl.ds(s * RPT, RPT)])


def _mm1_body(x_ref, w_ref, o_ref):
    o_ref[...] = jnp.dot(x_ref[...], w_ref[...],
                         preferred_element_type=jnp.float32)


def _mm1(x, w):
    return pl.pallas_call(
        _mm1_body,
        grid=(GRID,),
        in_specs=[pl.BlockSpec((BLK, DIN), lambda i: (i, 0)),
                  pl.BlockSpec((DIN, H), lambda i: (0, 0))],
        out_specs=pl.BlockSpec((BLK, H), lambda i: (i, 0)),
        out_shape=jax.ShapeDtypeStruct((NPAD, H), jnp.float32),
    )(x, w)


def _scale_body(degp_ref, h_ref, dinv_ref, hp_ref):
    d = degp_ref[0:1, :] + degp_ref[1:2, :] + 1.0
    dinv = lax.rsqrt(jnp.maximum(d, 1e-12))
    rep = lax.dot_general(dinv, jnp.ones((1, H), jnp.float32),
                          (((0,), (0,)), ((), ())),
                          preferred_element_type=jnp.float32)
    dinv_ref[...] = rep
    hp_ref[...] = h_ref[...] * rep


def _scale(degp, h):
    return pl.pallas_call(
        _scale_body,
        grid=(GRID,),
        in_specs=[pl.BlockSpec((NC, BLK), lambda i: (0, i)),
                  pl.BlockSpec((BLK, H), lambda i: (i, 0))],
        out_specs=[pl.BlockSpec((BLK, H), lambda i: (i, 0)),
                   pl.BlockSpec((BLK, H), lambda i: (i, 0))],
        out_shape=[jax.ShapeDtypeStruct((NPAD, H), jnp.float32),
                   jax.ShapeDtypeStruct((NPAD, H), jnp.float32)],
    )(degp, h)


def _layer2_body(p_ref, hp_ref, dinv_ref, b1_ref, w2_ref, o_ref):
    agg = dinv_ref[...] * (p_ref[0] + p_ref[1] + hp_ref[...])
    z = jax.nn.sigmoid(agg + b1_ref[...])
    o_ref[...] = jnp.dot(z, w2_ref[...],
                         preferred_element_type=jnp.float32) * dinv_ref[...]


def _layer2(p, hp, dinv, b1, w2):
    return pl.pallas_call(
        _layer2_body,
        grid=(GRID,),
        in_specs=[pl.BlockSpec((NC, BLK, H), lambda i: (0, i, 0)),
                  pl.BlockSpec((BLK, H), lambda i: (i, 0)),
                  pl.BlockSpec((BLK, H), lambda i: (i, 0)),
                  pl.BlockSpec((1, H), lambda i: (0, 0)),
                  pl.BlockSpec((H, H), lambda i: (0, 0))],
        out_specs=pl.BlockSpec((BLK, H), lambda i: (i, 0)),
        out_shape=jax.ShapeDtypeStruct((NPAD, H), jnp.float32),
    )(p, hp, dinv, b1, w2)


def _final_body(q_ref, hp_ref, dinv_ref, b2_ref, wm1_ref, bm1_ref,
                wm2_ref, bm2_ref, o_ref):
    agg = dinv_ref[...] * (q_ref[0] + q_ref[1] + hp_ref[...])
    z = jax.nn.sigmoid(agg + b2_ref[...])
    m = jnp.maximum(
        jnp.dot(z, wm1_ref[...], preferred_element_type=jnp.float32)
        + bm1_ref[...], 0.0)
    o_ref[...] = jnp.dot(m, wm2_ref[...],
                         preferred_element_type=jnp.float32) + bm2_ref[...]


def _final(q, hp, dinv, b2, wm1, bm1, wm2, bm2):
    return pl.pallas_call(
        _final_body,
        grid=(GRID,),
        in_specs=[pl.BlockSpec((NC, BLK, H), lambda i: (0, i, 0)),
                  pl.BlockSpec((BLK, H), lambda i: (i, 0)),
                  pl.BlockSpec((BLK, H), lambda i: (i, 0)),
                  pl.BlockSpec((1, H), lambda i: (0, 0)),
                  pl.BlockSpec((H, HM), lambda i: (0, 0)),
                  pl.BlockSpec((1, HM), lambda i: (0, 0)),
                  pl.BlockSpec((HM, OUT), lambda i: (0, 0)),
                  pl.BlockSpec((1, OUT), lambda i: (0, 0))],
        out_specs=pl.BlockSpec((BLK, OUT), lambda i: (i, 0)),
        out_shape=jax.ShapeDtypeStruct((NPAD, OUT), jnp.float32),
    )(q, hp, dinv, b2, wm1, bm1, wm2, bm2)


def kernel(features, edge_index, W1, b1, W2, b2, Wm1, bm1, Wm2, bm2):
    f32 = jnp.float32
    x = jnp.pad(features.astype(f32), ((0, NPAD - N), (0, 0)))
    src = edge_index[0].astype(jnp.int32)
    dst = edge_index[1].astype(jnp.int32)
    ar = jnp.arange(EPAD - E, dtype=jnp.int32)
    src = jnp.concatenate([src, ar % N])
    dst = jnp.concatenate([dst, N + ar % (NPAD - N)])
    src2d = src.reshape(IDX_ROWS, 128)
    dst2d = dst.reshape(IDX_ROWS, 128)
    zdeg = jnp.zeros((RPT,), f32)
    zrow = jnp.zeros((RPT, H), f32)

    degp = _deg_kernel(dst2d, zdeg)
    h1 = _mm1(x, W1.astype(f32))
    dinv_rep, h1p = _scale(degp, h1)
    p = _prop_kernel(h1p, src2d, dst2d, zrow)
    h2p = _layer2(p, h1p, dinv_rep, b1.reshape(1, H).astype(f32), W2.astype(f32))
    q = _prop_kernel(h2p, src2d, dst2d, zrow)
    out = _final(q, h2p, dinv_rep, b2.reshape(1, H).astype(f32),
                 Wm1.astype(f32), bm1.reshape(1, HM).astype(f32),
                 Wm2.astype(f32), bm2.reshape(1, OUT).astype(f32))
    return out[:N]


# speedup vs baseline: 38.0977x; 1.1850x over previous
import functools

import jax
import jax.numpy as jnp
from jax import lax
from jax.experimental import pallas as pl
from jax.experimental.pallas import tpu as pltpu
from jax.experimental.pallas import tpu_sc as plsc

N = 10000
NPAD = 10240
E = 320000
EPAD = 327680
DIN = 128
H = 64
HM = 128
OUT = 40
NC = 2
NS = 16
NW = NC * NS
CH = 512
CHUNKS = EPAD // (NW * CH)
IDX_ROWS = EPAD // 128
ROWS_PT = IDX_ROWS // NW
RPC = CH // 128
RPT = NPAD // NS
BLK = 512
GRID = NPAD // BLK

_mesh = plsc.VectorSubcoreMesh(core_axis_name="c", subcore_axis_name="s")


@functools.partial(
    pl.kernel,
    out_type=jax.ShapeDtypeStruct((NC, NPAD), jnp.float32),
    mesh=_mesh,
    scratch_types=[
        pltpu.VMEM((ROWS_PT, 128), jnp.int32),
        pltpu.VMEM((128,), jnp.float32),
        pltpu.VMEM_SHARED((NPAD,), jnp.float32),
    ],
)
def _deg_kernel(dst_hbm, zero_hbm, out_hbm, idx_v, ones_v, acc):
    c = lax.axis_index("c")
    s = lax.axis_index("s")
    wid = c * NS + s
    for i in range(8):
        ones_v[pl.ds(i * 16, 16)] = jnp.ones((16,), jnp.float32)
    pltpu.sync_copy(zero_hbm, acc.at[pl.ds(s * RPT, RPT)])
    pltpu.sync_copy(dst_hbm.at[pl.ds(wid * ROWS_PT, ROWS_PT)], idx_v)
    plsc.subcore_barrier()

    def body(j, carry):
        pltpu.sync_copy(ones_v, acc.at[idx_v.at[j]], add=True)
        return carry

    lax.fori_loop(0, ROWS_PT, body, 0)
    plsc.subcore_barrier()
    pltpu.sync_copy(acc.at[pl.ds(s * RPT, RPT)],
                    out_hbm.at[c, pl.ds(s * RPT, RPT)])


@functools.partial(
    pl.kernel,
    out_type=jax.ShapeDtypeStruct((NC, NPAD, H), jnp.float32),
    mesh=_mesh,
    compiler_params=pltpu.CompilerParams(use_tc_tiling_on_sc=False),
    scratch_types=[
        pltpu.VMEM((ROWS_PT, 128), jnp.int32),
        pltpu.VMEM((ROWS_PT, 128), jnp.int32),
        pltpu.VMEM((2, CH, H), jnp.float32),
        pltpu.VMEM_SHARED((NPAD, H), jnp.float32),
        pltpu.SemaphoreType.DMA,
        pltpu.SemaphoreType.DMA,
    ],
)
def _prop_kernel(hp_hbm, src_hbm, dst_hbm, zrow_hbm, out_hbm,
                 src_v, dst_v, rows_v, acc, sem0, sem1):
    c = lax.axis_index("c")
    s = lax.axis_index("s")
    wid = c * NS + s
    sems = (sem0, sem1)
    pltpu.sync_copy(zrow_hbm, acc.at[pl.ds(s * RPT, RPT)])
    pltpu.sync_copy(src_hbm.at[pl.ds(wid * ROWS_PT, ROWS_PT)], src_v)
    pltpu.sync_copy(dst_hbm.at[pl.ds(wid * ROWS_PT, ROWS_PT)], dst_v)
    plsc.subcore_barrier()

    def gather_start(k, b):
        base = k * RPC
        for j in range(RPC):
            pltpu.async_copy(hp_hbm.at[src_v.at[base + j]],
                             rows_v.at[b].at[pl.ds(j * 128, 128)], sems[b])

    def gather_wait(k, b):
        base = k * RPC
        for j in range(RPC):
            pltpu.make_async_copy(hp_hbm.at[src_v.at[base + j]],
                                  rows_v.at[b].at[pl.ds(j * 128, 128)],
                                  sems[b]).wait()

    def scatter(k, b):
        base = k * RPC
        for j in range(RPC):
            pltpu.sync_copy(rows_v.at[b].at[pl.ds(j * 128, 128)],
                            acc.at[dst_v.at[base + j]], add=True)

    gather_start(0, 0)

    def two_chunks(k0, carry):
        k = 2 * k0
        gather_wait(k, 0)
        gather_start(k + 1, 1)
        scatter(k, 0)
        gather_wait(k + 1, 1)
        gather_start(k + 2, 0)
        scatter(k + 1, 1)
        return carry

    lax.fori_loop(0, CHUNKS // 2 - 1, two_chunks, 0)
    k = CHUNKS - 2
    gather_wait(k, 0)
    gather_start(k + 1, 1)
    scatter(k, 0)
    gather_wait(k + 1, 1)
    scatter(k + 1, 1)
    plsc.subcore_barrier()
    pltpu.sync_copy(acc.at[pl.ds(s * RPT, RPT)],
                    out_hbm.at[c, pl.ds(s * RPT, RPT)])


def _mm1_body(degp_ref, x_ref, w_ref, dinv_ref, hp_ref):
    d = degp_ref[0:1, :] + degp_ref[1:2, :] + 1.0
    dinv = lax.rsqrt(jnp.maximum(d, 1e-12))
    rep = lax.dot_general(dinv, jnp.ones((1, H), jnp.float32),
                          (((0,), (0,)), ((), ())),
                          preferred_element_type=jnp.float32)
    dinv_ref[...] = rep
    hp_ref[...] = jnp.dot(x_ref[...], w_ref[...],
                          preferred_element_type=jnp.float32) * rep


def _mm1(degp, x, w):
    return pl.pallas_call(
        _mm1_body,
        grid=(GRID,),
        in_specs=[pl.BlockSpec((NC, BLK), lambda i: (0, i)),
                  pl.BlockSpec((BLK, DIN), lambda i: (i, 0)),
                  pl.BlockSpec((DIN, H), lambda i: (0, 0))],
        out_specs=[pl.BlockSpec((BLK, H), lambda i: (i, 0)),
                   pl.BlockSpec((BLK, H), lambda i: (i, 0))],
        out_shape=[jax.ShapeDtypeStruct((NPAD, H), jnp.float32),
                   jax.ShapeDtypeStruct((NPAD, H), jnp.float32)],
    )(degp, x, w)


def _layer2_body(p_ref, hp_ref, dinv_ref, b1_ref, w2_ref, o_ref):
    agg = dinv_ref[...] * (p_ref[0] + p_ref[1] + hp_ref[...])
    z = jax.nn.sigmoid(agg + b1_ref[...])
    o_ref[...] = jnp.dot(z, w2_ref[...],
                         preferred_element_type=jnp.float32) * dinv_ref[...]


def _layer2(p, hp, dinv, b1, w2):
    return pl.pallas_call(
        _layer2_body,
        grid=(GRID,),
        in_specs=[pl.BlockSpec((NC, BLK, H), lambda i: (0, i, 0)),
                  pl.BlockSpec((BLK, H), lambda i: (i, 0)),
                  pl.BlockSpec((BLK, H), lambda i: (i, 0)),
                  pl.BlockSpec((1, H), lambda i: (0, 0)),
                  pl.BlockSpec((H, H), lambda i: (0, 0))],
        out_specs=pl.BlockSpec((BLK, H), lambda i: (i, 0)),
        out_shape=jax.ShapeDtypeStruct((NPAD, H), jnp.float32),
    )(p, hp, dinv, b1, w2)


def _final_body(q_ref, hp_ref, dinv_ref, b2_ref, wm1_ref, bm1_ref,
                wm2_ref, bm2_ref, o_ref):
    agg = dinv_ref[...] * (q_ref[0] + q_ref[1] + hp_ref[...])
    z = jax.nn.sigmoid(agg + b2_ref[...])
    m = jnp.maximum(
        jnp.dot(z, wm1_ref[...], preferred_element_type=jnp.float32)
        + bm1_ref[...], 0.0)
    o_ref[...] = jnp.dot(m, wm2_ref[...],
                         preferred_element_type=jnp.float32) + bm2_ref[...]


def _final(q, hp, dinv, b2, wm1, bm1, wm2, bm2):
    return pl.pallas_call(
        _final_body,
        grid=(GRID,),
        in_specs=[pl.BlockSpec((NC, BLK, H), lambda i: (0, i, 0)),
                  pl.BlockSpec((BLK, H), lambda i: (i, 0)),
                  pl.BlockSpec((BLK, H), lambda i: (i, 0)),
                  pl.BlockSpec((1, H), lambda i: (0, 0)),
                  pl.BlockSpec((H, HM), lambda i: (0, 0)),
                  pl.BlockSpec((1, HM), lambda i: (0, 0)),
                  pl.BlockSpec((HM, OUT), lambda i: (0, 0)),
                  pl.BlockSpec((1, OUT), lambda i: (0, 0))],
        out_specs=pl.BlockSpec((BLK, OUT), lambda i: (i, 0)),
        out_shape=jax.ShapeDtypeStruct((NPAD, OUT), jnp.float32),
    )(q, hp, dinv, b2, wm1, bm1, wm2, bm2)


def kernel(features, edge_index, W1, b1, W2, b2, Wm1, bm1, Wm2, bm2):
    f32 = jnp.float32
    x = jnp.pad(features.astype(f32), ((0, NPAD - N), (0, 0)))
    src = edge_index[0].astype(jnp.int32)
    dst = edge_index[1].astype(jnp.int32)
    ar = jnp.arange(EPAD - E, dtype=jnp.int32)
    src = jnp.concatenate([src, ar % N])
    dst = jnp.concatenate([dst, N + ar % (NPAD - N)])
    src2d = src.reshape(IDX_ROWS, 128)
    dst2d = dst.reshape(IDX_ROWS, 128)
    zdeg = jnp.zeros((RPT,), f32)
    zrow = jnp.zeros((RPT, H), f32)

    degp = _deg_kernel(dst2d, zdeg)
    dinv_rep, h1p = _mm1(degp, x, W1.astype(f32))
    p = _prop_kernel(h1p, src2d, dst2d, zrow)
    h2p = _layer2(p, h1p, dinv_rep, b1.reshape(1, H).astype(f32), W2.astype(f32))
    q = _prop_kernel(h2p, src2d, dst2d, zrow)
    out = _final(q, h2p, dinv_rep, b2.reshape(1, H).astype(f32),
                 Wm1.astype(f32), bm1.reshape(1, HM).astype(f32),
                 Wm2.astype(f32), bm2.reshape(1, OUT).astype(f32))
    return out[:N]


# speedup vs baseline: 45.9644x; 1.2065x over previous
import functools

import jax
import jax.numpy as jnp
from jax import lax
from jax.experimental import pallas as pl
from jax.experimental.pallas import tpu as pltpu
from jax.experimental.pallas import tpu_sc as plsc

N = 10000
NPADD = 10240
E = 320000
DIN = 128
H = 64
HM = 128
OUT = 40
NC = 2
NS = 16
NW = NC * NS
ER = E // 128
RT = 78
EXB = NW * RT
FULL = 19
RPC = 4
RPT = NPADD // NS
BLK = 2048
GRID = 5

_mesh = plsc.VectorSubcoreMesh(core_axis_name="c", subcore_axis_name="s")


@functools.partial(
    pl.kernel,
    out_type=jax.ShapeDtypeStruct((NC, NPADD), jnp.float32),
    mesh=_mesh,
    compiler_params=pltpu.CompilerParams(use_tc_tiling_on_sc=False),
    scratch_types=[
        pltpu.VMEM((RT + 2, 128), jnp.int32),
        pltpu.VMEM((128,), jnp.float32),
        pltpu.VMEM_SHARED((NPADD,), jnp.float32),
        pltpu.SemaphoreType.DMA,
    ],
)
def _deg_kernel(e2d_hbm, zero_hbm, out_hbm, idx_v, ones_v, acc, sem):
    c = lax.axis_index("c")
    s = lax.axis_index("s")
    wid = c * NS + s
    for i in range(8):
        ones_v[pl.ds(i * 16, 16)] = jnp.ones((16,), jnp.float32)
    pltpu.sync_copy(e2d_hbm.at[pl.ds(ER + wid * RT, RT)],
                    idx_v.at[pl.ds(0, RT)])

    @pl.when(s == 0)
    def _():
        pltpu.sync_copy(e2d_hbm.at[pl.ds(ER + EXB + 2 * c, 2)],
                        idx_v.at[pl.ds(RT, 2)])

    pltpu.sync_copy(zero_hbm.at[pl.ds(s * RPT, RPT)],
                    acc.at[pl.ds(s * RPT, RPT)])
    plsc.subcore_barrier()

    def body(j, carry):
        cps = [
            pltpu.async_copy(ones_v, acc.at[idx_v.at[j * 6 + i]], sem,
                             add=True)
            for i in range(6)
        ]
        for cp in cps:
            cp.wait()
        return carry

    lax.fori_loop(0, RT // 6, body, 0)

    @pl.when(s == 0)
    def _():
        for i in range(2):
            pltpu.async_copy(ones_v, acc.at[idx_v.at[RT + i]], sem,
                             add=True).wait()

    plsc.subcore_barrier()
    pltpu.sync_copy(acc.at[pl.ds(s * RPT, RPT)],
                    out_hbm.at[c, pl.ds(s * RPT, RPT)])


@functools.partial(
    pl.kernel,
    out_type=jax.ShapeDtypeStruct((NC, NPADD, H), jnp.float32),
    mesh=_mesh,
    compiler_params=pltpu.CompilerParams(use_tc_tiling_on_sc=False),
    scratch_types=[
        pltpu.VMEM((RT + 2, 128), jnp.int32),
        pltpu.VMEM((RT + 2, 128), jnp.int32),
        pltpu.VMEM((2, RPC * 128, H), jnp.float32),
        pltpu.VMEM_SHARED((NPADD, H), jnp.float32),
        pltpu.SemaphoreType.DMA,
        pltpu.SemaphoreType.DMA,
        pltpu.SemaphoreType.DMA,
        pltpu.SemaphoreType.DMA,
    ],
)
def _prop_kernel(hp_hbm, e2d_hbm, zrow_hbm, out_hbm,
                 src_v, dst_v, rows_v, acc, sem0, sem1, ssem0, ssem1):
    c = lax.axis_index("c")
    s = lax.axis_index("s")
    wid = c * NS + s
    gsems = (sem0, sem1)
    ssems = (ssem0, ssem1)
    pltpu.sync_copy(e2d_hbm.at[pl.ds(wid * RT, RT)], src_v.at[pl.ds(0, RT)])
    pltpu.sync_copy(e2d_hbm.at[pl.ds(ER + wid * RT, RT)],
                    dst_v.at[pl.ds(0, RT)])

    @pl.when(s == 0)
    def _():
        pltpu.sync_copy(e2d_hbm.at[pl.ds(EXB + 2 * c, 2)],
                        src_v.at[pl.ds(RT, 2)])
        pltpu.sync_copy(e2d_hbm.at[pl.ds(ER + EXB + 2 * c, 2)],
                        dst_v.at[pl.ds(RT, 2)])

    def gather_start(k, b):
        base = k * RPC
        for j in range(RPC):
            pltpu.async_copy(hp_hbm.at[src_v.at[base + j]],
                             rows_v.at[b].at[pl.ds(j * 128, 128)], gsems[b])

    def gather_wait(k, b):
        base = k * RPC
        for j in range(RPC):
            pltpu.make_async_copy(hp_hbm.at[src_v.at[base + j]],
                                  rows_v.at[b].at[pl.ds(j * 128, 128)],
                                  gsems[b]).wait()

    def scatter_start(k, b):
        base = k * RPC
        return [
            pltpu.async_copy(rows_v.at[b].at[pl.ds(j * 128, 128)],
                             acc.at[dst_v.at[base + j]], ssems[b], add=True)
            for j in range(RPC)
        ]

    gather_start(0, 0)
    pltpu.sync_copy(zrow_hbm.at[pl.ds(s * RPT, RPT)],
                    acc.at[pl.ds(s * RPT, RPT)])
    plsc.subcore_barrier()

    def two_chunks(k0, carry):
        k = 2 * k0
        gather_wait(k, 0)
        gather_start(k + 1, 1)
        sc0 = scatter_start(k, 0)
        gather_wait(k + 1, 1)
        sc1 = scatter_start(k + 1, 1)
        for cp in sc0:
            cp.wait()
        gather_start(k + 2, 0)
        for cp in sc1:
            cp.wait()
        return carry

    lax.fori_loop(0, FULL // 2 - 1, two_chunks, 0)
    two_chunks(jnp.int32(FULL // 2 - 1), 0)
    k = FULL - 1
    gather_wait(k, 0)
    for j in range(2):
        pltpu.async_copy(hp_hbm.at[src_v.at[FULL * RPC + j]],
                         rows_v.at[1].at[pl.ds(j * 128, 128)], sem1)
    sc0 = scatter_start(k, 0)
    for j in range(2):
        pltpu.make_async_copy(hp_hbm.at[src_v.at[FULL * RPC + j]],
                              rows_v.at[1].at[pl.ds(j * 128, 128)],
                              sem1).wait()
    sc1 = [
        pltpu.async_copy(rows_v.at[1].at[pl.ds(j * 128, 128)],
                         acc.at[dst_v.at[FULL * RPC + j]], ssem1, add=True)
        for j in range(2)
    ]
    for cp in sc0 + sc1:
        cp.wait()

    @pl.when(s == 0)
    def _():
        for j in range(2):
            pltpu.async_copy(hp_hbm.at[src_v.at[RT + j]],
                             rows_v.at[0].at[pl.ds(j * 128, 128)],
                             sem0).wait()
            pltpu.async_copy(rows_v.at[0].at[pl.ds(j * 128, 128)],
                             acc.at[dst_v.at[RT + j]], ssem0,
                             add=True).wait()

    plsc.subcore_barrier()
    pltpu.sync_copy(acc.at[pl.ds(s * RPT, RPT)],
                    out_hbm.at[c, pl.ds(s * RPT, RPT)])


def _mm1_body(degp_ref, x_ref, w_ref, dinv_ref, hp_ref):
    d = degp_ref[0:1, :] + degp_ref[1:2, :] + 1.0
    dinv = lax.rsqrt(jnp.maximum(d, 1e-12))
    rep = lax.dot_general(dinv, jnp.ones((1, H), jnp.float32),
                          (((0,), (0,)), ((), ())),
                          preferred_element_type=jnp.float32)
    dinv_ref[...] = rep
    hp_ref[...] = jnp.dot(x_ref[...], w_ref[...],
                          preferred_element_type=jnp.float32) * rep


def _mm1(degp, x, w):
    return pl.pallas_call(
        _mm1_body,
        grid=(GRID,),
        in_specs=[pl.BlockSpec((NC, BLK), lambda i: (0, i)),
                  pl.BlockSpec((BLK, DIN), lambda i: (i, 0)),
                  pl.BlockSpec((DIN, H), lambda i: (0, 0))],
        out_specs=[pl.BlockSpec((BLK, H), lambda i: (i, 0)),
                   pl.BlockSpec((BLK, H), lambda i: (i, 0))],
        out_shape=[jax.ShapeDtypeStruct((N, H), jnp.float32),
                   jax.ShapeDtypeStruct((N, H), jnp.float32)],
    )(degp, x, w)


def _layer2_body(p_ref, hp_ref, dinv_ref, b1_ref, w2_ref, o_ref):
    agg = dinv_ref[...] * (p_ref[0] + p_ref[1] + hp_ref[...])
    z = jax.nn.sigmoid(agg + b1_ref[...])
    o_ref[...] = jnp.dot(z, w2_ref[...],
                         preferred_element_type=jnp.float32) * dinv_ref[...]


def _layer2(p, hp, dinv, b1, w2):
    return pl.pallas_call(
        _layer2_body,
        grid=(GRID,),
        in_specs=[pl.BlockSpec((NC, BLK, H), lambda i: (0, i, 0)),
                  pl.BlockSpec((BLK, H), lambda i: (i, 0)),
                  pl.BlockSpec((BLK, H), lambda i: (i, 0)),
                  pl.BlockSpec((1, H), lambda i: (0, 0)),
                  pl.BlockSpec((H, H), lambda i: (0, 0))],
        out_specs=pl.BlockSpec((BLK, H), lambda i: (i, 0)),
        out_shape=jax.ShapeDtypeStruct((N, H), jnp.float32),
    )(p, hp, dinv, b1, w2)


def _final_body(q_ref, hp_ref, dinv_ref, b2_ref, wm1_ref, bm1_ref,
                wm2_ref, bm2_ref, o_ref):
    agg = dinv_ref[...] * (q_ref[0] + q_ref[1] + hp_ref[...])
    z = jax.nn.sigmoid(agg + b2_ref[...])
    m = jnp.maximum(
        jnp.dot(z, wm1_ref[...], preferred_element_type=jnp.float32)
        + bm1_ref[...], 0.0)
    o_ref[...] = jnp.dot(m, wm2_ref[...],
                         preferred_element_type=jnp.float32) + bm2_ref[...]


def _final(q, hp, dinv, b2, wm1, bm1, wm2, bm2):
    return pl.pallas_call(
        _final_body,
        grid=(GRID,),
        in_specs=[pl.BlockSpec((NC, BLK, H), lambda i: (0, i, 0)),
                  pl.BlockSpec((BLK, H), lambda i: (i, 0)),
                  pl.BlockSpec((BLK, H), lambda i: (i, 0)),
                  pl.BlockSpec((1, H), lambda i: (0, 0)),
                  pl.BlockSpec((H, HM), lambda i: (0, 0)),
                  pl.BlockSpec((1, HM), lambda i: (0, 0)),
                  pl.BlockSpec((HM, OUT), lambda i: (0, 0)),
                  pl.BlockSpec((1, OUT), lambda i: (0, 0))],
        out_specs=pl.BlockSpec((BLK, OUT), lambda i: (i, 0)),
        out_shape=jax.ShapeDtypeStruct((N, OUT), jnp.float32),
    )(q, hp, dinv, b2, wm1, bm1, wm2, bm2)


def kernel(features, edge_index, W1, b1, W2, b2, Wm1, bm1, Wm2, bm2):
    f32 = jnp.float32
    x = features.astype(f32)
    e2d = edge_index.astype(jnp.int32).reshape(2 * ER, 128)
    zdeg = jnp.zeros((NPADD,), f32)
    zrow = jnp.zeros((NPADD, H), f32)

    degp = _deg_kernel(e2d, zdeg)
    dinv_rep, h1p = _mm1(degp, x, W1.astype(f32))
    p = _prop_kernel(h1p, e2d, zrow)
    h2p = _layer2(p, h1p, dinv_rep, b1.reshape(1, H).astype(f32),
                  W2.astype(f32))
    q = _prop_kernel(h2p, e2d, zrow)
    out = _final(q, h2p, dinv_rep, b2.reshape(1, H).astype(f32),
                 Wm1.astype(f32), bm1.reshape(1, HM).astype(f32),
                 Wm2.astype(f32), bm2.reshape(1, OUT).astype(f32))
    return out
